# baseline (speedup 1.0000x reference)
import functools

import jax
import jax.numpy as jnp
from jax import lax
from jax.experimental import pallas as pl
from jax.experimental.pallas import tpu as pltpu
from jax.experimental.pallas import tpu_sc as plsc

N = 10000
E = 320000
D = 128

NC = 2
NS = 16
NW = NC * NS
EPW = E // NW
K = 128
EPWP = -(-EPW // K) * K
PAD = EPWP - EPW
NCHUNK = EPWP // K
NA = N + 8
NBUF = 2
STRIPE = 640
LAST = N - (NS - 1) * STRIPE
VT = jnp.float32

_MESH = plsc.VectorSubcoreMesh(core_axis_name="c", subcore_axis_name="s")


def _seg_body(with_deg, *refs):
    if with_deg:
        (y_hbm, src_hbm, dst_hbm, z2_hbm, agg_hbm, deg_hbm,
         s0_v, s1_v, d0_v, d1_v, r0, r1, ones_v, degtmp_v,
         acc_sh, deg_sh,
         sg0, sg1, ss0, ss1, sl0, sl1, sm0, sm1, sd) = refs
    else:
        (y_hbm, src_hbm, dst_hbm, z2_hbm, agg_hbm,
         s0_v, s1_v, d0_v, d1_v, r0, r1,
         acc_sh,
         sg0, sg1, ss0, ss1, sl0, sl1, sm0, sm1) = refs
    rows = (r0, r1)
    srcb = (s0_v, s1_v)
    dstb = (d0_v, d1_v)
    sg = (sg0, sg1)
    ss = (ss0, ss1)
    sl = (sl0, sl1)
    sm = (sm0, sm1)

    cid = lax.axis_index("c")
    sid = lax.axis_index("s")
    wid = sid * NC + cid
    roff = sid * STRIPE

    pltpu.async_copy(src_hbm.at[wid, 0], srcb[0], sm[0])
    pltpu.async_copy(src_hbm.at[wid, 1], srcb[1], sm[1])
    pltpu.async_copy(dst_hbm.at[wid, 0], dstb[0], sl[0])

    @pl.when(sid < NS - 1)
    def _():
        pltpu.sync_copy(z2_hbm.at[pl.ds(roff, STRIPE)],
                        acc_sh.at[pl.ds(roff, STRIPE)])

    @pl.when(sid == NS - 1)
    def _():
        pltpu.sync_copy(z2_hbm.at[pl.ds(roff, LAST)],
                        acc_sh.at[pl.ds(roff, LAST)])

    if with_deg:
        for j in range(K // 16):
            ones_v[pl.ds(j * 16, 16)] = jnp.ones((16,), jnp.float32)
        for j in range(STRIPE // 16):
            degtmp_v[pl.ds(j * 16, 16)] = jnp.zeros((16,), jnp.float32)

        @pl.when(sid < NS - 1)
        def _():
            pltpu.sync_copy(degtmp_v, deg_sh.at[pl.ds(roff, STRIPE)])

        @pl.when(sid == NS - 1)
        def _():
            pltpu.sync_copy(degtmp_v.at[pl.ds(0, LAST)],
                            deg_sh.at[pl.ds(roff, LAST)])

    pltpu.make_async_copy(src_hbm.at[wid, 0], srcb[0], sm[0]).wait()
    pltpu.async_copy(y_hbm.at[srcb[0]], rows[0], sg[0])

    plsc.subcore_barrier()

    def gwait(r):
        pltpu.make_async_copy(y_hbm.at[srcb[r]], rows[r], sg[r]).wait()

    def swait(r):
        pltpu.make_async_copy(rows[r], acc_sh.at[dstb[r]], ss[r]).wait()

    def lwait(r):
        pltpu.make_async_copy(dst_hbm.at[wid, 0], dstb[r], sl[r]).wait()

    def mwait(r):
        pltpu.make_async_copy(src_hbm.at[wid, 0], srcb[r], sm[r]).wait()

    def dwait():
        pltpu.make_async_copy(ones_v, deg_sh.at[dstb[0]], sd).wait()

    def chunk(a, r):
        rr = 1 - r

        @pl.when(a >= 1)
        def _():
            swait(rr)
            if with_deg:
                dwait()

        @pl.when(a + 1 < NCHUNK)
        def _():
            pltpu.async_copy(dst_hbm.at[wid, a + 1], dstb[rr], sl[rr])
            mwait(rr)
            pltpu.async_copy(y_hbm.at[srcb[rr]], rows[rr], sg[rr])

        gwait(r)

        @pl.when(a + 2 < NCHUNK)
        def _():
            pltpu.async_copy(src_hbm.at[wid, a + 2], srcb[r], sm[r])

        lwait(r)
        pltpu.async_copy(rows[r], acc_sh.at[dstb[r]], ss[r], add=True)
        if with_deg:
            pltpu.async_copy(ones_v, deg_sh.at[dstb[r]], sd, add=True)

    def body2(j, carry):
        a0 = j * NBUF
        for r in range(NBUF):
            chunk(a0 + r, r)
        return carry

    lax.fori_loop(0, NCHUNK // NBUF, body2, 0)
    chunk(NCHUNK - 1, (NCHUNK - 1) % NBUF)
    swait(0)
    if with_deg:
        dwait()

    plsc.subcore_barrier()

    @pl.when(sid < NS - 1)
    def _():
        pltpu.sync_copy(acc_sh.at[pl.ds(roff, STRIPE)],
                        agg_hbm.at[cid, pl.ds(roff, STRIPE)])
        if with_deg:
            pltpu.sync_copy(deg_sh.at[pl.ds(roff, STRIPE)], degtmp_v)
            pltpu.sync_copy(degtmp_v,
                            deg_hbm.at[pl.ds(cid * N + roff, STRIPE)])

    @pl.when(sid == NS - 1)
    def _():
        pltpu.sync_copy(acc_sh.at[pl.ds(roff, LAST)],
                        agg_hbm.at[cid, pl.ds(roff, LAST)])
        if with_deg:
            pltpu.sync_copy(deg_sh.at[pl.ds(roff, LAST)],
                            degtmp_v.at[pl.ds(0, LAST)])
            pltpu.sync_copy(degtmp_v.at[pl.ds(0, LAST)],
                            deg_hbm.at[pl.ds(cid * N + roff, LAST)])


_row_bufs = [pltpu.VMEM((K, D), VT)] * NBUF
_idx_bufs = [pltpu.VMEM((K,), jnp.int32)] * (2 * NBUF)
_sems = [pltpu.SemaphoreType.DMA] * (4 * NBUF)

_seg1 = pl.kernel(
    functools.partial(_seg_body, True),
    out_type=(jax.ShapeDtypeStruct((NC, N, D), VT),
              jax.ShapeDtypeStruct((NC * N,), jnp.float32)),
    mesh=_MESH,
    scratch_types=[
        *_idx_bufs,
        *_row_bufs,
        pltpu.VMEM((K,), jnp.float32),
        pltpu.VMEM((STRIPE,), jnp.float32),
        pltpu.VMEM_SHARED((NA, D), VT),
        pltpu.VMEM_SHARED((NA,), jnp.float32),
        *_sems,
        pltpu.SemaphoreType.DMA,
    ],
)

_seg2 = pl.kernel(
    functools.partial(_seg_body, False),
    out_type=jax.ShapeDtypeStruct((NC, N, D), VT),
    mesh=_MESH,
    scratch_types=[
        *_idx_bufs,
        *_row_bufs,
        pltpu.VMEM_SHARED((NA, D), VT),
        *_sems,
    ],
)


BM = 1000
GRID = N // BM

_rowspec = pl.BlockSpec((BM, D), lambda i: (i, 0))
_wspec = pl.BlockSpec((D, D), lambda i: (0, 0))
_dspec = pl.BlockSpec((BM, 1), lambda i: (i, 0))
_bspec = pl.BlockSpec((1, D), lambda i: (0, 0))


def _mm_body(x_ref, w_ref, y_ref):
    y = jnp.dot(x_ref[...], w_ref[...], preferred_element_type=jnp.float32)
    y_ref[...] = y.astype(VT)


_mm = pl.pallas_call(
    _mm_body,
    grid=(GRID,),
    in_specs=[_rowspec, _wspec],
    out_specs=_rowspec,
    out_shape=jax.ShapeDtypeStruct((N, D), VT),
)


def _mid_body(xs_ref, a0_ref, a1_ref, d0_ref, d1_ref, b_ref,
              wn_ref, y_ref, h_ref):
    deg = jnp.maximum(d0_ref[...] + d1_ref[...], 1.0)
    agg = a0_ref[...].astype(jnp.float32) + a1_ref[...].astype(jnp.float32)
    h = xs_ref[...] + agg / deg + b_ref[...]
    h = jnp.maximum(h, 0.0)
    y = jnp.dot(h, wn_ref[...], preferred_element_type=jnp.float32)
    y_ref[...] = y.astype(VT)
    h_ref[...] = h


_mid = pl.pallas_call(
    _mid_body,
    grid=(GRID,),
    in_specs=[_rowspec, _rowspec, _rowspec, _dspec, _dspec, _bspec,
              _wspec],
    out_specs=[_rowspec, _rowspec],
    out_shape=[jax.ShapeDtypeStruct((N, D), VT),
               jax.ShapeDtypeStruct((N, D), jnp.float32)],
)


def _fin_body(hs_ref, a0_ref, a1_ref, d0_ref, d1_ref, b_ref, o_ref):
    deg = jnp.maximum(d0_ref[...] + d1_ref[...], 1.0)
    agg = a0_ref[...].astype(jnp.float32) + a1_ref[...].astype(jnp.float32)
    o_ref[...] = hs_ref[...] + agg / deg + b_ref[...]


_fin = pl.pallas_call(
    _fin_body,
    grid=(GRID,),
    in_specs=[_rowspec, _rowspec, _rowspec, _dspec, _dspec, _bspec],
    out_specs=_rowspec,
    out_shape=jax.ShapeDtypeStruct((N, D), jnp.float32),
)


def kernel(features, edge_index, W1_self, W1_neigh, b1, W2_self, W2_neigh, b2):
    srcw = edge_index[0].reshape(NW, EPW)
    dstw = edge_index[1].reshape(NW, EPW)
    src3 = jnp.pad(srcw, ((0, 0), (0, PAD))).reshape(NW, NCHUNK, K)
    dst3 = jnp.pad(dstw, ((0, 0), (0, PAD)),
                   constant_values=N).reshape(NW, NCHUNK, K)
    z2 = jnp.zeros((N, D), VT)

    y1 = _mm(features, W1_neigh)
    aggp1, degp = _seg1(y1, src3, dst3, z2)
    xs1 = _mm(features, W1_self)
    degp = degp.reshape(NC, N)
    d0 = degp[0][:, None]
    d1 = degp[1][:, None]
    y2, h1 = _mid(xs1, aggp1[0], aggp1[1], d0, d1, b1[None, :], W2_neigh)
    aggp2 = _seg2(y2, src3, dst3, z2)
    hs2 = _mm(h1, W2_self)
    return _fin(hs2, aggp2[0], aggp2[1], d0, d1, b2[None, :])


# speedup vs baseline: 1.7210x; 1.7210x over previous
import functools

import jax
import jax.numpy as jnp
from jax import lax
from jax.experimental import pallas as pl
from jax.experimental.pallas import tpu as pltpu
from jax.experimental.pallas import tpu_sc as plsc

N = 10000
E = 320000
D = 128

NC = 2
NS = 16
NW = NC * NS
EPW = E // NW
K = 80
NCHUNK = EPW // K
NBUF = 2
STRIPE = 640
LAST = N - (NS - 1) * STRIPE
VT = jnp.float32

_MESH = plsc.VectorSubcoreMesh(core_axis_name="c", subcore_axis_name="s")


def _seg_body(with_deg, *refs):
    if with_deg:
        (y_hbm, src_hbm, dst_hbm, z2_hbm, agg_hbm, deg_hbm,
         src_all, d0_v, d1_v, r0, r1, ones_v, degtmp_v,
         acc_sh, deg_sh,
         sg0, sg1, ss0, ss1, sl0, sl1, sd, si) = refs
    else:
        (y_hbm, src_hbm, dst_hbm, z2_hbm, agg_hbm,
         src_all, d0_v, d1_v, r0, r1,
         acc_sh,
         sg0, sg1, ss0, ss1, sl0, sl1, si) = refs
    rows = (r0, r1)
    dstb = (d0_v, d1_v)
    sg = (sg0, sg1)
    ss = (ss0, ss1)
    sl = (sl0, sl1)

    cid = lax.axis_index("c")
    sid = lax.axis_index("s")
    wid = sid * NC + cid
    roff = sid * STRIPE

    pltpu.async_copy(src_hbm.at[wid], src_all, si)
    pltpu.async_copy(dst_hbm.at[wid, 0], dstb[0], sl[0])

    @pl.when(sid < NS - 1)
    def _():
        pltpu.sync_copy(z2_hbm.at[pl.ds(roff, STRIPE)],
                        acc_sh.at[pl.ds(roff, STRIPE)])

    @pl.when(sid == NS - 1)
    def _():
        pltpu.sync_copy(z2_hbm.at[pl.ds(roff, LAST)],
                        acc_sh.at[pl.ds(roff, LAST)])

    if with_deg:
        for j in range(K // 16):
            ones_v[pl.ds(j * 16, 16)] = jnp.ones((16,), jnp.float32)
        for j in range(STRIPE // 16):
            degtmp_v[pl.ds(j * 16, 16)] = jnp.zeros((16,), jnp.float32)

        @pl.when(sid < NS - 1)
        def _():
            pltpu.sync_copy(degtmp_v, deg_sh.at[pl.ds(roff, STRIPE)])

        @pl.when(sid == NS - 1)
        def _():
            pltpu.sync_copy(degtmp_v.at[pl.ds(0, LAST)],
                            deg_sh.at[pl.ds(roff, LAST)])

    pltpu.make_async_copy(src_hbm.at[wid], src_all, si).wait()

    pltpu.async_copy(y_hbm.at[src_all.at[0]], rows[0], sg[0])

    plsc.subcore_barrier()

    def gwait(r):
        pltpu.make_async_copy(y_hbm.at[src_all.at[0]], rows[r], sg[r]).wait()

    def swait(r):
        pltpu.make_async_copy(rows[r], acc_sh.at[dstb[r]], ss[r]).wait()

    def lwait(r):
        pltpu.make_async_copy(dst_hbm.at[wid, 0], dstb[r], sl[r]).wait()

    def dwait():
        pltpu.make_async_copy(ones_v, deg_sh.at[dstb[0]], sd).wait()

    def chunk(a, r):
        rr = 1 - r

        @pl.when(a >= 1)
        def _():
            swait(rr)
            if with_deg:
                dwait()

        @pl.when(a + 1 < NCHUNK)
        def _():
            pltpu.async_copy(y_hbm.at[src_all.at[a + 1]], rows[rr], sg[rr])
            pltpu.async_copy(dst_hbm.at[wid, a + 1], dstb[rr], sl[rr])

        gwait(r)
        lwait(r)
        pltpu.async_copy(rows[r], acc_sh.at[dstb[r]], ss[r], add=True)
        if with_deg:
            pltpu.async_copy(ones_v, deg_sh.at[dstb[r]], sd, add=True)

    def body2(j, carry):
        a0 = j * NBUF
        for r in range(NBUF):
            chunk(a0 + r, r)
        return carry

    lax.fori_loop(0, NCHUNK // NBUF, body2, 0)
    chunk(NCHUNK - 1, (NCHUNK - 1) % NBUF)
    swait(0)
    if with_deg:
        dwait()

    plsc.subcore_barrier()

    @pl.when(sid < NS - 1)
    def _():
        pltpu.sync_copy(acc_sh.at[pl.ds(roff, STRIPE)],
                        agg_hbm.at[cid, pl.ds(roff, STRIPE)])
        if with_deg:
            pltpu.sync_copy(deg_sh.at[pl.ds(roff, STRIPE)], degtmp_v)
            pltpu.sync_copy(degtmp_v,
                            deg_hbm.at[pl.ds(cid * N + roff, STRIPE)])

    @pl.when(sid == NS - 1)
    def _():
        pltpu.sync_copy(acc_sh.at[pl.ds(roff, LAST)],
                        agg_hbm.at[cid, pl.ds(roff, LAST)])
        if with_deg:
            pltpu.sync_copy(deg_sh.at[pl.ds(roff, LAST)],
                            degtmp_v.at[pl.ds(0, LAST)])
            pltpu.sync_copy(degtmp_v.at[pl.ds(0, LAST)],
                            deg_hbm.at[pl.ds(cid * N + roff, LAST)])


_row_bufs = [pltpu.VMEM((K, D), VT)] * NBUF
_idx_bufs = [pltpu.VMEM((NCHUNK, K), jnp.int32),
             pltpu.VMEM((K,), jnp.int32),
             pltpu.VMEM((K,), jnp.int32)]
_sems = [pltpu.SemaphoreType.DMA] * (3 * NBUF)

_seg1 = pl.kernel(
    functools.partial(_seg_body, True),
    out_type=(jax.ShapeDtypeStruct((NC, N, D), VT),
              jax.ShapeDtypeStruct((NC * N,), jnp.float32)),
    mesh=_MESH,
    scratch_types=[
        *_idx_bufs,
        *_row_bufs,
        pltpu.VMEM((K,), jnp.float32),
        pltpu.VMEM((STRIPE,), jnp.float32),
        pltpu.VMEM_SHARED((N, D), VT),
        pltpu.VMEM_SHARED((N,), jnp.float32),
        *_sems,
        pltpu.SemaphoreType.DMA,
        pltpu.SemaphoreType.DMA,
    ],
)

_seg2 = pl.kernel(
    functools.partial(_seg_body, False),
    out_type=jax.ShapeDtypeStruct((NC, N, D), VT),
    mesh=_MESH,
    scratch_types=[
        *_idx_bufs,
        *_row_bufs,
        pltpu.VMEM_SHARED((N, D), VT),
        *_sems,
        pltpu.SemaphoreType.DMA,
    ],
)


BM = 1000
GRID = N // BM

_rowspec = pl.BlockSpec((BM, D), lambda i: (i, 0))
_wspec = pl.BlockSpec((D, D), lambda i: (0, 0))
_dspec = pl.BlockSpec((BM, 1), lambda i: (i, 0))
_bspec = pl.BlockSpec((1, D), lambda i: (0, 0))


def _mm_body(x_ref, w_ref, y_ref):
    y = jnp.dot(x_ref[...], w_ref[...], preferred_element_type=jnp.float32)
    y_ref[...] = y.astype(VT)


_mm = pl.pallas_call(
    _mm_body,
    grid=(GRID,),
    in_specs=[_rowspec, _wspec],
    out_specs=_rowspec,
    out_shape=jax.ShapeDtypeStruct((N, D), VT),
)


def _mid_body(xs_ref, a0_ref, a1_ref, d0_ref, d1_ref, b_ref,
              wn_ref, y_ref, h_ref):
    deg = jnp.maximum(d0_ref[...] + d1_ref[...], 1.0)
    agg = a0_ref[...].astype(jnp.float32) + a1_ref[...].astype(jnp.float32)
    h = xs_ref[...] + agg / deg + b_ref[...]
    h = jnp.maximum(h, 0.0)
    y = jnp.dot(h, wn_ref[...], preferred_element_type=jnp.float32)
    y_ref[...] = y.astype(VT)
    h_ref[...] = h


_mid = pl.pallas_call(
    _mid_body,
    grid=(GRID,),
    in_specs=[_rowspec, _rowspec, _rowspec, _dspec, _dspec, _bspec,
              _wspec],
    out_specs=[_rowspec, _rowspec],
    out_shape=[jax.ShapeDtypeStruct((N, D), VT),
               jax.ShapeDtypeStruct((N, D), jnp.float32)],
)


def _fin_body(hs_ref, a0_ref, a1_ref, d0_ref, d1_ref, b_ref, o_ref):
    deg = jnp.maximum(d0_ref[...] + d1_ref[...], 1.0)
    agg = a0_ref[...].astype(jnp.float32) + a1_ref[...].astype(jnp.float32)
    o_ref[...] = hs_ref[...] + agg / deg + b_ref[...]


_fin = pl.pallas_call(
    _fin_body,
    grid=(GRID,),
    in_specs=[_rowspec, _rowspec, _rowspec, _dspec, _dspec, _bspec],
    out_specs=_rowspec,
    out_shape=jax.ShapeDtypeStruct((N, D), jnp.float32),
)


def kernel(features, edge_index, W1_self, W1_neigh, b1, W2_self, W2_neigh, b2):
    src3 = edge_index[0].reshape(NW, NCHUNK, K)
    dst3 = edge_index[1].reshape(NW, NCHUNK, K)
    z2 = jnp.zeros((N, D), VT)

    y1 = _mm(features, W1_neigh)
    aggp1, degp = _seg1(y1, src3, dst3, z2)
    xs1 = _mm(features, W1_self)
    degp = degp.reshape(NC, N)
    d0 = degp[0][:, None]
    d1 = degp[1][:, None]
    y2, h1 = _mid(xs1, aggp1[0], aggp1[1], d0, d1, b1[None, :], W2_neigh)
    aggp2 = _seg2(y2, src3, dst3, z2)
    hs2 = _mm(h1, W2_self)
    return _fin(hs2, aggp2[0], aggp2[1], d0, d1, b2[None, :])
